# baseline (speedup 1.0000x reference)
import jax
import jax.numpy as jnp
from jax import lax
from jax.experimental import pallas as pl
from jax.experimental.pallas import tpu as pltpu

N_TOK = 4096
REP_DIM = 1024
Z_DIM = 256
N_DIGITS = 16
TB = 256


def _body(dig_ref, rep_ref, w_ref, out_ref):
    e = pl.program_id(1)
    zk = lax.dot_general(rep_ref[...], w_ref[0],
                         (((1,), (1,)), ((), ())),
                         preferred_element_type=jnp.float32)
    mask = dig_ref[...] == e

    @pl.when(e == 0)
    def _init():
        out_ref[...] = jnp.where(mask, zk, jnp.zeros_like(zk))

    @pl.when(e > 0)
    def _merge():
        out_ref[...] = jnp.where(mask, zk, out_ref[...])

    @pl.when(e == N_DIGITS - 1)
    def _expmap():
        z = out_ref[...]
        n = jnp.sqrt(jnp.sum(z * z, axis=-1, keepdims=True))
        n = jnp.maximum(n, 1e-15)
        out_ref[...] = jnp.tanh(n) * z / n


def kernel(rep, digits, W):
    dig2d = digits.reshape(N_TOK, 1)
    grid = (N_TOK // TB, N_DIGITS)
    return pl.pallas_call(
        _body,
        grid=grid,
        in_specs=[
            pl.BlockSpec((TB, 1), lambda i, e: (i, 0)),
            pl.BlockSpec((TB, REP_DIM), lambda i, e: (i, 0)),
            pl.BlockSpec((1, Z_DIM, REP_DIM), lambda i, e: (e, 0, 0)),
        ],
        out_specs=pl.BlockSpec((TB, Z_DIM), lambda i, e: (i, 0)),
        out_shape=jax.ShapeDtypeStruct((N_TOK, Z_DIM), jnp.float32),
    )(dig2d, rep, W)


# speedup vs baseline: 2.5575x; 2.5575x over previous
import jax
import jax.numpy as jnp
from jax import lax
from jax.experimental import pallas as pl
from jax.experimental.pallas import tpu as pltpu
import jax.experimental.pallas.tpu_sc as plsc

N_TOK = 4096
REP_DIM = 1024
Z_DIM = 256
NE = 16
BLK = 128
CAP = N_TOK + NE * BLK
NB = CAP // BLK
NC, NS = 2, 16
NW = NC * NS
T = N_TOK // NW
TV = T // 16


def _dyng(x, idx):
    return lax.gather(
        x, idx[:, None],
        lax.GatherDimensionNumbers(offset_dims=(), collapsed_slice_dims=(0,),
                                   start_index_map=(0,)),
        (1,), mode=lax.GatherScatterMode.PROMISE_IN_BOUNDS)


def _lane_iota():
    return lax.iota(jnp.int32, 16)


def _splat_sum(x):
    lane = _lane_iota()
    for k in (8, 4, 2, 1):
        x = x + _dyng(x, lane ^ k)
    return x


def _cumsum16(x):
    lane = _lane_iota()
    for k in (1, 2, 4, 8):
        shifted = _dyng(x, jnp.maximum(lane - k, 0))
        x = x + jnp.where(lane >= k, shifted, 0)
    return x


def _lane_splat(x, e):
    return _dyng(x, jnp.full((16,), e, jnp.int32))


def _hist_body(dig_hbm, hist_hbm, digc_v, histloc_v):
    wid = lax.axis_index("s") * NC + lax.axis_index("c")
    tb = wid * T
    lane = _lane_iota()
    pltpu.sync_copy(dig_hbm.at[pl.ds(tb, T)], digc_v)
    vs = [digc_v[pl.ds(j * 16, 16)] for j in range(TV)]
    hist = jnp.zeros(16, jnp.int32)
    for e in range(NE):
        s = jnp.zeros(16, jnp.int32)
        for v in vs:
            s = s + jnp.where(v == e, 1, 0)
        s = _splat_sum(s)
        hist = hist + jnp.where(lane == e, s, 0)
    histloc_v[...] = hist
    pltpu.sync_copy(histloc_v, hist_hbm.at[wid])


def _route_body(dig_hbm, rep_hbm, hist_hbm, reps_hbm, slot_hbm, be_hbm,
                dig_v, h_v, pos_v, posf_v, bev_v, rowbuf_v,
                sem_in0, sem_in1, sem_out0, sem_out1):
    wid = lax.axis_index("s") * NC + lax.axis_index("c")
    tb = wid * T
    lane = lax.iota(jnp.int32, 16)

    pltpu.sync_copy(dig_hbm.at[pl.ds(tb, T)], dig_v)
    pltpu.sync_copy(hist_hbm, h_v)

    total = jnp.zeros(16, jnp.int32)
    pref = jnp.zeros(16, jnp.int32)
    for i in range(NW):
        r = h_v[i, :]
        total = total + r
        su = jnp.minimum(jnp.maximum(wid - i, 0), 1)
        pref = pref + r * su

    nblk = lax.shift_right_logical(total + (BLK - 1), BLK.bit_length() - 1)
    cum = _cumsum16(nblk)
    base = (cum - nblk) * BLK
    nxt = base + pref

    for j in range(TV):
        v = dig_v[pl.ds(j * 16, 16)]
        pos = jnp.zeros(16, jnp.int32)
        for e in range(NE):
            mi = jnp.where(v == e, 1, 0)
            c = _cumsum16(mi)
            cnt = _lane_splat(c, 15)
            ne_v = _lane_splat(nxt, e)
            pos = pos + jnp.where(v == e, ne_v + c - 1, 0)
            nxt = nxt + jnp.where(lane == e, cnt, 0)
        pos_v[j, :] = pos
        posf_v[pl.ds(j * 16, 16)] = pos

    pltpu.sync_copy(posf_v, slot_hbm.at[wid])

    @pl.when(wid == 0)
    def _blkexp():
        for i in range(NB // 16):
            bstart = (lane + i * 16) * BLK
            cnt = jnp.zeros(16, jnp.int32)
            for e in range(NE):
                be_v = _lane_splat(base, e)
                cnt = cnt + jnp.where(be_v <= bstart, 1, 0)
            bev_v[pl.ds(i * 16, 16)] = cnt - 1
        pltpu.sync_copy(bev_v, be_hbm)

    sems_in = (sem_in0, sem_in1)
    sems_out = (sem_out0, sem_out1)

    def start_in(j):
        return pltpu.async_copy(rep_hbm.at[pl.ds(tb + j * 16, 16)],
                                rowbuf_v.at[j % 2], sems_in[j % 2])

    def start_out(j):
        return pltpu.async_copy(rowbuf_v.at[j % 2],
                                reps_hbm.at[pos_v.at[j]], sems_out[j % 2])

    in_copies = [None, None]
    out_copies = [None, None]
    in_copies[0] = start_in(0)
    for j in range(TV):
        nj = j + 1
        if nj < TV:
            if out_copies[nj % 2] is not None:
                out_copies[nj % 2].wait()
            in_copies[nj % 2] = start_in(nj)
        in_copies[j % 2].wait()
        out_copies[j % 2] = start_out(j)
    out_copies[(TV - 2) % 2].wait()
    out_copies[(TV - 1) % 2].wait()


def _mm_body(be_ref, rep_ref, w_ref, out_ref):
    z = lax.dot_general(rep_ref[...], w_ref[0], (((1,), (1,)), ((), ())),
                        preferred_element_type=jnp.float32)
    n = jnp.sqrt(jnp.sum(z * z, axis=-1, keepdims=True))
    n = jnp.maximum(n, 1e-15)
    out_ref[...] = jnp.tanh(n) * z / n


def _gather_body(slot_hbm, zf_hbm, out_hbm, slot_v, rows_v, sem):
    wid = lax.axis_index("s") * NC + lax.axis_index("c")
    pltpu.sync_copy(slot_hbm.at[wid], slot_v)
    pltpu.async_copy(zf_hbm.at[slot_v], rows_v, sem).wait()
    pltpu.sync_copy(rows_v, out_hbm.at[pl.ds(wid * T, T)])


def _make_mesh():
    return plsc.VectorSubcoreMesh(core_axis_name="c", subcore_axis_name="s",
                                  num_cores=NC, num_subcores=NS)


def kernel(rep, digits, W):
    mesh = _make_mesh()
    hist_k = pl.kernel(
        _hist_body,
        out_type=jax.ShapeDtypeStruct((NW, 16), jnp.int32),
        mesh=mesh,
        scratch_types=[
            pltpu.VMEM((T,), jnp.int32),
            pltpu.VMEM((16,), jnp.int32),
        ],
    )
    hist = hist_k(digits)

    route = pl.kernel(
        _route_body,
        out_type=(jax.ShapeDtypeStruct((CAP, REP_DIM), jnp.float32),
                  jax.ShapeDtypeStruct((NW, T), jnp.int32),
                  jax.ShapeDtypeStruct((NB,), jnp.int32)),
        mesh=mesh,
        scratch_types=[
            pltpu.VMEM((T,), jnp.int32),
            pltpu.VMEM((NW, 16), jnp.int32),
            pltpu.VMEM((TV, 16), jnp.int32),
            pltpu.VMEM((T,), jnp.int32),
            pltpu.VMEM((NB,), jnp.int32),
            pltpu.VMEM((2, 16, REP_DIM), jnp.float32),
            pltpu.SemaphoreType.DMA,
            pltpu.SemaphoreType.DMA,
            pltpu.SemaphoreType.DMA,
            pltpu.SemaphoreType.DMA,
        ],
    )
    reps, slot, be = route(digits, rep, hist)

    zf = pl.pallas_call(
        _mm_body,
        grid_spec=pltpu.PrefetchScalarGridSpec(
            num_scalar_prefetch=1,
            grid=(NB,),
            in_specs=[
                pl.BlockSpec((BLK, REP_DIM), lambda b, be_r: (b, 0)),
                pl.BlockSpec((1, Z_DIM, REP_DIM), lambda b, be_r: (be_r[b], 0, 0)),
            ],
            out_specs=pl.BlockSpec((BLK, Z_DIM), lambda b, be_r: (b, 0)),
        ),
        out_shape=jax.ShapeDtypeStruct((CAP, Z_DIM), jnp.float32),
    )(be, reps, W)

    gather = pl.kernel(
        _gather_body,
        out_type=jax.ShapeDtypeStruct((N_TOK, Z_DIM), jnp.float32),
        mesh=mesh,
        scratch_types=[
            pltpu.VMEM((T,), jnp.int32),
            pltpu.VMEM((T, Z_DIM), jnp.float32),
            pltpu.SemaphoreType.DMA,
        ],
    )
    return gather(slot, zf)


# speedup vs baseline: 2.6630x; 1.0413x over previous
import jax
import jax.numpy as jnp
from jax import lax
from jax.experimental import pallas as pl
from jax.experimental.pallas import tpu as pltpu
import jax.experimental.pallas.tpu_sc as plsc

N_TOK = 4096
REP_DIM = 1024
Z_DIM = 256
NE = 16
BLK = 128
CAP = N_TOK + NE * BLK
NB = CAP // BLK
NC, NS = 2, 16
NW = NC * NS
T = N_TOK // NW
TV = T // 16
HT = N_TOK // NS


def _dyng(x, idx):
    return lax.gather(
        x, idx[:, None],
        lax.GatherDimensionNumbers(offset_dims=(), collapsed_slice_dims=(0,),
                                   start_index_map=(0,)),
        (1,), mode=lax.GatherScatterMode.PROMISE_IN_BOUNDS)


def _lane_iota():
    return lax.iota(jnp.int32, 16)


def _splat_sum(x):
    lane = _lane_iota()
    for k in (8, 4, 2, 1):
        x = x + _dyng(x, lane ^ k)
    return x


def _cumsum16(x):
    lane = _lane_iota()
    for k in (1, 2, 4, 8):
        shifted = _dyng(x, jnp.maximum(lane - k, 0))
        x = x + jnp.where(lane >= k, shifted, 0)
    return x


def _lane_splat(x, e):
    return _dyng(x, jnp.full((16,), e, jnp.int32))


def _hist16(vregs):
    lane = _lane_iota()
    hist = jnp.zeros(16, jnp.int32)
    for e in range(NE):
        s = jnp.zeros(16, jnp.int32)
        for v in vregs:
            s = s + jnp.where(v == e, 1, 0)
        s = _splat_sum(s)
        hist = hist + jnp.where(lane == e, s, 0)
    return hist


def _route_body(dig_hbm, rep_hbm, reps_hbm, slot_hbm, be_hbm, hx_hbm,
                digh_v, histloc2_v, h_v, dig_v, pos_v, posf_v, bev_v,
                rowbuf_v, sem_in0, sem_in1, sem_out0, sem_out1):
    cid = lax.axis_index("c")
    sid = lax.axis_index("s")
    wid = sid * NC + cid
    tb = wid * T
    lane = _lane_iota()

    pltpu.sync_copy(dig_hbm.at[pl.ds(sid * HT, HT)], digh_v)
    vsh = [digh_v[pl.ds(j * 16, 16)] for j in range(HT // 16)]
    histloc2_v[0, :] = _hist16(vsh[:T // 16])
    histloc2_v[1, :] = _hist16(vsh[T // 16:])
    pltpu.sync_copy(histloc2_v, hx_hbm.at[cid, pl.ds(2 * sid, 2)])
    plsc.subcore_barrier()
    pltpu.sync_copy(hx_hbm.at[cid], h_v)

    total = jnp.zeros(16, jnp.int32)
    pref = jnp.zeros(16, jnp.int32)
    for i in range(NW):
        r = h_v[i, :]
        total = total + r
        su = jnp.minimum(jnp.maximum(wid - i, 0), 1)
        pref = pref + r * su

    nblk = lax.shift_right_logical(total + (BLK - 1), BLK.bit_length() - 1)
    cum = _cumsum16(nblk)
    base = (cum - nblk) * BLK
    nxt = base + pref

    pltpu.sync_copy(dig_hbm.at[pl.ds(tb, T)], dig_v)
    for j in range(TV):
        v = dig_v[pl.ds(j * 16, 16)]
        pos = jnp.zeros(16, jnp.int32)
        for e in range(NE):
            mi = jnp.where(v == e, 1, 0)
            c = _cumsum16(mi)
            cnt = _lane_splat(c, 15)
            ne_v = _lane_splat(nxt, e)
            pos = pos + jnp.where(v == e, ne_v + c - 1, 0)
            nxt = nxt + jnp.where(lane == e, cnt, 0)
        pos_v[j, :] = pos
        posf_v[pl.ds(j * 16, 16)] = pos

    pltpu.sync_copy(posf_v, slot_hbm.at[wid])

    @pl.when(wid == 0)
    def _blkexp():
        for i in range(NB // 16):
            bstart = (lane + i * 16) * BLK
            cnt = jnp.zeros(16, jnp.int32)
            for e in range(NE):
                be_v = _lane_splat(base, e)
                cnt = cnt + jnp.where(be_v <= bstart, 1, 0)
            bev_v[pl.ds(i * 16, 16)] = cnt - 1
        pltpu.sync_copy(bev_v, be_hbm)

    sems_in = (sem_in0, sem_in1)
    sems_out = (sem_out0, sem_out1)

    def start_in(j):
        return pltpu.async_copy(rep_hbm.at[pl.ds(tb + j * 16, 16)],
                                rowbuf_v.at[j % 2], sems_in[j % 2])

    def start_out(j):
        return pltpu.async_copy(rowbuf_v.at[j % 2],
                                reps_hbm.at[pos_v.at[j]], sems_out[j % 2])

    in_copies = [None, None]
    out_copies = [None, None]
    in_copies[0] = start_in(0)
    for j in range(TV):
        nj = j + 1
        if nj < TV:
            if out_copies[nj % 2] is not None:
                out_copies[nj % 2].wait()
            in_copies[nj % 2] = start_in(nj)
        in_copies[j % 2].wait()
        out_copies[j % 2] = start_out(j)
    out_copies[(TV - 2) % 2].wait()
    out_copies[(TV - 1) % 2].wait()


def _mm_body(be_ref, rep_ref, w_ref, out_ref):
    z = lax.dot_general(rep_ref[...], w_ref[0], (((1,), (1,)), ((), ())),
                        preferred_element_type=jnp.float32)
    n = jnp.sqrt(jnp.sum(z * z, axis=-1, keepdims=True))
    n = jnp.maximum(n, 1e-15)
    out_ref[...] = jnp.tanh(n) * z / n


def _gather_body(slot_hbm, zf_hbm, out_hbm, slot_v, rows_v, sem):
    wid = lax.axis_index("s") * NC + lax.axis_index("c")
    pltpu.sync_copy(slot_hbm.at[wid], slot_v)
    pltpu.async_copy(zf_hbm.at[slot_v], rows_v, sem).wait()
    pltpu.sync_copy(rows_v, out_hbm.at[pl.ds(wid * T, T)])


def _make_mesh():
    return plsc.VectorSubcoreMesh(core_axis_name="c", subcore_axis_name="s",
                                  num_cores=NC, num_subcores=NS)


def kernel(rep, digits, W):
    mesh = _make_mesh()
    route = pl.kernel(
        _route_body,
        out_type=(jax.ShapeDtypeStruct((CAP, REP_DIM), jnp.float32),
                  jax.ShapeDtypeStruct((NW, T), jnp.int32),
                  jax.ShapeDtypeStruct((NB,), jnp.int32),
                  jax.ShapeDtypeStruct((NC, NW, 16), jnp.int32)),
        mesh=mesh,
        scratch_types=[
            pltpu.VMEM((HT,), jnp.int32),
            pltpu.VMEM((2, 16), jnp.int32),
            pltpu.VMEM((NW, 16), jnp.int32),
            pltpu.VMEM((T,), jnp.int32),
            pltpu.VMEM((TV, 16), jnp.int32),
            pltpu.VMEM((T,), jnp.int32),
            pltpu.VMEM((NB,), jnp.int32),
            pltpu.VMEM((2, 16, REP_DIM), jnp.float32),
            pltpu.SemaphoreType.DMA,
            pltpu.SemaphoreType.DMA,
            pltpu.SemaphoreType.DMA,
            pltpu.SemaphoreType.DMA,
        ],
    )
    reps, slot, be, _hx = route(digits, rep)

    zf = pl.pallas_call(
        _mm_body,
        grid_spec=pltpu.PrefetchScalarGridSpec(
            num_scalar_prefetch=1,
            grid=(NB,),
            in_specs=[
                pl.BlockSpec((BLK, REP_DIM), lambda b, be_r: (b, 0)),
                pl.BlockSpec((1, Z_DIM, REP_DIM), lambda b, be_r: (be_r[b], 0, 0)),
            ],
            out_specs=pl.BlockSpec((BLK, Z_DIM), lambda b, be_r: (b, 0)),
        ),
        out_shape=jax.ShapeDtypeStruct((CAP, Z_DIM), jnp.float32),
    )(be, reps, W)

    gather = pl.kernel(
        _gather_body,
        out_type=jax.ShapeDtypeStruct((N_TOK, Z_DIM), jnp.float32),
        mesh=mesh,
        scratch_types=[
            pltpu.VMEM((T,), jnp.int32),
            pltpu.VMEM((T, Z_DIM), jnp.float32),
            pltpu.SemaphoreType.DMA,
        ],
    )
    return gather(slot, zf)
